# baseline (speedup 1.0000x reference)
import functools

import jax
import jax.numpy as jnp
from jax import lax
from jax.experimental import pallas as pl
from jax.experimental.pallas import tpu as pltpu
from jax.experimental.pallas import tpu_sc as plsc

_N, _D = 8192, 2048
_NC, _NS, _L = 2, 16, 16
_NW = _NC * _NS
_RPW = _N // _NW
_K = 8
_NVR = _RPW // _L


def _sc_body(x_hbm, t_hbm, q_hbm, out_hbm,
             q_v, idx_v, qs_v, xb, tb, accb, sem1, sem2):
    wid = lax.axis_index("s") * _NC + lax.axis_index("c")
    base = wid * _RPW
    pltpu.sync_copy(q_hbm.at[pl.ds(base, _RPW)], q_v)

    for j in range(_NVR):
        sl = pl.ds(j * _L, _L)
        idx_v[sl] = jnp.full((_L,), base, jnp.int32)
        qs_v[sl] = jnp.zeros((_L,), jnp.int32)

    cnt = jnp.int32(0)
    for j in range(_NVR):
        qv = q_v[pl.ds(j * _L, _L)]
        m = qv != 0
        mi = jnp.where(m, 1, 0)
        pos = jnp.maximum(cnt + plsc.cumsum(mi) - 1, 0)
        rows = lax.iota(jnp.int32, _L) + (base + j * _L)
        plsc.store_scatter(idx_v, [pos], rows, mask=m)
        plsc.store_scatter(qs_v, [pos], qv, mask=m)
        cnt = cnt + jnp.sum(mi)

    nch = (cnt + _K - 1) // _K

    def chunk(c, acc):
        off = pl.multiple_of(c * _K, _K)
        idxs = idx_v.at[pl.ds(off, _K)]
        cp1 = pltpu.async_copy(x_hbm.at[idxs], xb, sem1)
        cp2 = pltpu.async_copy(t_hbm.at[idxs], tb, sem2)
        cp1.wait()
        cp2.wait()
        for k in range(_K):
            qb = plsc.load_gather(qs_v, [jnp.full((_L,), off + k, jnp.int32)])
            m1 = qb == 1
            g2 = jnp.where(qb == 2, 1.0, 0.0)

            def inner(v, a):
                x = xb[k, pl.ds(v * _L, _L)]
                t = tb[k, pl.ds(v * _L, _L)]
                d = x - t
                w = jnp.where(m1, t, g2)
                g = w * d
                return a + g * g

            acc = lax.fori_loop(0, _D // _L, inner, acc)
        return acc

    acc = lax.fori_loop(0, nch, chunk, jnp.zeros((_L,), jnp.float32))
    accb[...] = acc
    pltpu.sync_copy(accb, out_hbm.at[wid])


def _sc_partials(input, target, q):
    mesh = plsc.VectorSubcoreMesh(core_axis_name="c", subcore_axis_name="s")
    f = functools.partial(
        pl.kernel,
        mesh=mesh,
        compiler_params=pltpu.CompilerParams(needs_layout_passes=False),
        out_type=jax.ShapeDtypeStruct((_NW, _L), jnp.float32),
        scratch_types=[
            pltpu.VMEM((_RPW,), jnp.int32),
            pltpu.VMEM((_RPW,), jnp.int32),
            pltpu.VMEM((_RPW,), jnp.int32),
            pltpu.VMEM((_K, _D), jnp.float32),
            pltpu.VMEM((_K, _D), jnp.float32),
            pltpu.VMEM((_L,), jnp.float32),
            pltpu.SemaphoreType.DMA,
            pltpu.SemaphoreType.DMA,
        ],
    )(_sc_body)
    return f(input, target, q)


def kernel(input, target, q):
    partials = _sc_partials(input, target, q)
    return jnp.sum(partials) / (_N * _D)


# speedup vs baseline: 2.1712x; 2.1712x over previous
import functools

import jax
import jax.numpy as jnp
from jax import lax
from jax.experimental import pallas as pl
from jax.experimental.pallas import tpu as pltpu
from jax.experimental.pallas import tpu_sc as plsc

_N, _D = 8192, 2048
_NC, _NS, _L = 2, 16, 16
_NW = _NC * _NS
_RPW = _N // _NW
_K = 8
_NVR = _RPW // _L
_MAXCH = _RPW // _K
_IPAD = _RPW + 2 * _K
_UNROLL = 8
_VPR = _D // _L


def _sc_body(x_hbm, t_hbm, q_hbm, out_hbm,
             q_v, idx_v, qs_v, xb0, tb0, xb1, tb1, accb,
             sx0, st0, sx1, st1):
    wid = lax.axis_index("s") * _NC + lax.axis_index("c")
    base = wid * _RPW
    pltpu.sync_copy(q_hbm.at[pl.ds(base, _RPW)], q_v)

    for j in range(_IPAD // _L):
        sl = pl.ds(j * _L, _L)
        idx_v[sl] = jnp.full((_L,), base, jnp.int32)
        qs_v[sl] = jnp.zeros((_L,), jnp.int32)

    cnt = jnp.int32(0)
    for j in range(_NVR):
        qv = q_v[pl.ds(j * _L, _L)]
        m = qv != 0
        mi = jnp.where(m, 1, 0)
        pos = jnp.maximum(cnt + plsc.cumsum(mi) - 1, 0)
        rows = lax.iota(jnp.int32, _L) + (base + j * _L)
        plsc.store_scatter(idx_v, [pos], rows, mask=m)
        plsc.store_scatter(qs_v, [pos], qv, mask=m)
        cnt = cnt + jnp.sum(mi)

    nch = (cnt + _K - 1) // _K
    npair = (nch + 1) // 2

    bufs = ((xb0, tb0, sx0, st0), (xb1, tb1, sx1, st1))

    def issue(c, b):
        off = pl.multiple_of(c * _K, _K)
        idxs = idx_v.at[pl.ds(off, _K)]
        xb, tb, sx, st = bufs[b]
        pltpu.async_copy(x_hbm.at[idxs], xb, sx)
        pltpu.async_copy(t_hbm.at[idxs], tb, st)

    def wait(b):
        xb, tb, sx, st = bufs[b]
        pltpu.make_async_copy(x_hbm.at[idx_v.at[pl.ds(0, _K)]], xb, sx).wait()
        pltpu.make_async_copy(t_hbm.at[idx_v.at[pl.ds(0, _K)]], tb, st).wait()

    def compute(c, b, acc):
        off = pl.multiple_of(c * _K, _K)
        xb, tb, _, _ = bufs[b]
        for k in range(_K):
            qb = plsc.load_gather(qs_v, [jnp.full((_L,), off + k, jnp.int32)])
            m1 = qb == 1
            g2 = jnp.where(qb == 2, 1.0, 0.0)

            def inner(v, a, _k=k, _m1=m1, _g2=g2):
                for u in range(_UNROLL):
                    sl = pl.ds((v * _UNROLL + u) * _L, _L)
                    x = xb[_k, sl]
                    t = tb[_k, sl]
                    d = x - t
                    w = jnp.where(_m1, t, _g2)
                    g = w * d
                    a = a + g * g
                return a

            acc = lax.fori_loop(0, _VPR // _UNROLL, inner, acc)
        return acc

    issue(0, 0)

    def pair(i2, acc):
        c0 = i2 * 2
        issue(c0 + 1, 1)
        wait(0)
        acc = compute(c0, 0, acc)
        issue(c0 + 2, 0)
        wait(1)
        acc = compute(c0 + 1, 1, acc)
        return acc

    acc = lax.fori_loop(0, npair, pair, jnp.zeros((_L,), jnp.float32))
    wait(0)

    accb[...] = acc
    pltpu.sync_copy(accb, out_hbm.at[wid])


def _sc_partials(input, target, q):
    mesh = plsc.VectorSubcoreMesh(core_axis_name="c", subcore_axis_name="s")
    f = functools.partial(
        pl.kernel,
        mesh=mesh,
        compiler_params=pltpu.CompilerParams(needs_layout_passes=False),
        out_type=jax.ShapeDtypeStruct((_NW, _L), jnp.float32),
        scratch_types=[
            pltpu.VMEM((_RPW,), jnp.int32),
            pltpu.VMEM((_IPAD,), jnp.int32),
            pltpu.VMEM((_IPAD,), jnp.int32),
            pltpu.VMEM((_K, _D), jnp.float32),
            pltpu.VMEM((_K, _D), jnp.float32),
            pltpu.VMEM((_K, _D), jnp.float32),
            pltpu.VMEM((_K, _D), jnp.float32),
            pltpu.VMEM((_L,), jnp.float32),
            pltpu.SemaphoreType.DMA,
            pltpu.SemaphoreType.DMA,
            pltpu.SemaphoreType.DMA,
            pltpu.SemaphoreType.DMA,
        ],
    )(_sc_body)
    return f(input, target, q)


def kernel(input, target, q):
    partials = _sc_partials(input, target, q)
    return jnp.sum(partials) / (_N * _D)
